# baseline (speedup 1.0000x reference)
import functools

import jax
import jax.numpy as jnp
from jax import lax
from jax.experimental import pallas as pl
from jax.experimental.pallas import tpu as pltpu
from jax.experimental.pallas import tpu_sc as plsc

N_NODES = 10000
N_EDGES = 320000
D = 128

NC = 2
NS = 16
NW = NC * NS
NP = 10240
RPT = NP // NS
CH = 128
EPAD = 327680
NCHT = EPAD // CH
TCH = NCHT // NS
NCH_C0 = 160
NCH_C1 = TCH - NCH_C0
DCH = EPAD // (NC * NS) // CH

_mesh = plsc.VectorSubcoreMesh(core_axis_name="c", subcore_axis_name="s")


@functools.partial(
    pl.kernel,
    out_type=jax.ShapeDtypeStruct((NC, NP), jnp.float32),
    mesh=_mesh,
    scratch_types=[
        pltpu.VMEM((DCH, CH), jnp.int32),
        pltpu.VMEM((CH,), jnp.float32),
        pltpu.VMEM((RPT,), jnp.float32),
        pltpu.VMEM_SHARED((NP,), jnp.float32),
        pltpu.SemaphoreType.DMA,
    ],
)
def _deg(dst4_hbm, zeros1_hbm, ones_hbm, deg_hbm, dstall, onesv, degv, deg_sh,
         sem):
    c = lax.axis_index("c")
    s = lax.axis_index("s")
    base_r = s * RPT
    pltpu.sync_copy(zeros1_hbm, degv)
    pltpu.sync_copy(degv, deg_sh.at[pl.ds(base_r, RPT)])
    pltpu.sync_copy(ones_hbm, onesv)
    pltpu.sync_copy(dst4_hbm.at[c, s], dstall)
    plsc.subcore_barrier()

    k = 8

    def group(g, carry):
        for b in range(k):
            pltpu.async_copy(onesv, deg_sh.at[dstall.at[g * k + b]], sem,
                             add=True)
        for b in range(k):
            pltpu.make_async_copy(onesv, deg_sh.at[dstall.at[g * k + b]],
                                  sem).wait()
        return carry

    lax.fori_loop(0, DCH // k, group, 0)
    plsc.subcore_barrier()

    pltpu.sync_copy(deg_sh.at[pl.ds(base_r, RPT)], degv)
    pltpu.sync_copy(degv, deg_hbm.at[c, pl.ds(base_r, RPT)])


@functools.partial(
    pl.kernel,
    out_type=jax.ShapeDtypeStruct((NC, NP, D), jnp.float32),
    mesh=_mesh,
    scratch_types=[
        pltpu.VMEM((4, CH), jnp.int32),
        pltpu.VMEM((4, CH), jnp.int32),
        pltpu.VMEM((2, CH, D), jnp.float32),
        pltpu.VMEM_SHARED((NP, D), jnp.float32),
        pltpu.SemaphoreType.DMA,
        pltpu.SemaphoreType.DMA,
        pltpu.SemaphoreType.DMA,
        pltpu.SemaphoreType.DMA,
        pltpu.SemaphoreType.DMA,
        pltpu.SemaphoreType.DMA,
    ],
)
def _gather_scatter(src2_hbm, dst2_hbm, g_hbm, zeros2_hbm, out_hbm,
                    srcv, dstv, rows, acc_sh,
                    si0, si1, si2, si3, sg0, sg1):
    c = lax.axis_index("c")
    s = lax.axis_index("s")
    base_r = s * RPT
    base_ch = jnp.where(c == 0, s * NCH_C0, NS * NCH_C0 + s * NCH_C1)
    nch = jnp.where(c == 0, NCH_C0, NCH_C1)
    sis = (si0, si1, si2, si3)
    sgs = (sg0, sg1)

    def start_idx(i, q):
        pltpu.async_copy(src2_hbm.at[base_ch + i], srcv.at[q], sis[q])
        pltpu.async_copy(dst2_hbm.at[base_ch + i], dstv.at[q], sis[q])

    def wait_idx(i, q):
        pltpu.make_async_copy(src2_hbm.at[base_ch + i], srcv.at[q], sis[q]).wait()
        pltpu.make_async_copy(dst2_hbm.at[base_ch + i], dstv.at[q], sis[q]).wait()

    def start_gather(i, q, r):
        pltpu.async_copy(g_hbm.at[srcv.at[q]], rows.at[r], sgs[r])

    def wait_gather(i, q, r):
        pltpu.make_async_copy(g_hbm.at[srcv.at[q]], rows.at[r], sgs[r]).wait()

    @pl.when(nch > 0)
    def _():
        start_idx(0, 0)

    @pl.when(nch > 1)
    def _():
        start_idx(1, 1)

    @pl.when(nch > 2)
    def _():
        start_idx(2, 2)

    pltpu.sync_copy(zeros2_hbm, acc_sh.at[pl.ds(base_r, RPT)])
    plsc.subcore_barrier()

    @pl.when(nch > 0)
    def _():
        wait_idx(0, 0)
        start_gather(0, 0, 0)

    def quad(gq, carry):
        for b in range(4):
            i = gq * 4 + b
            wait_gather(i, b, b % 2)

            @pl.when(i + 1 < nch)
            def _():
                wait_idx(i + 1, (b + 1) % 4)
                start_gather(i + 1, (b + 1) % 4, (b + 1) % 2)

            @pl.when(i + 3 < nch)
            def _():
                start_idx(i + 3, (b + 3) % 4)

            pltpu.sync_copy(rows.at[b % 2], acc_sh.at[dstv.at[b]], add=True)
        return carry

    lax.fori_loop(0, nch // 4, quad, 0)
    plsc.subcore_barrier()

    pltpu.sync_copy(acc_sh.at[pl.ds(base_r, RPT)], out_hbm.at[c, pl.ds(base_r, RPT)])


_RB = 2000
_NB = N_NODES // _RB

_row_spec = pl.BlockSpec((_RB, D), lambda i: (i, 0))
_col_spec = pl.BlockSpec((_RB, 1), lambda i: (i, 0))
_w_spec = pl.BlockSpec((D, D), lambda i: (0, 0))
_b_spec = pl.BlockSpec((1, D), lambda i: (0, 0))


def _mm_scale_body(x_ref, w_ref, deg0_ref, deg1_ref, g_ref, dinv_ref):
    dinv = lax.rsqrt(deg0_ref[...] + deg1_ref[...] + 1.0)
    dinv_ref[...] = dinv
    h = jnp.dot(x_ref[...], w_ref[...], preferred_element_type=jnp.float32)
    g_ref[...] = h * dinv


_mm_scale = pl.pallas_call(
    _mm_scale_body,
    grid=(_NB,),
    in_specs=[_row_spec, _w_spec, _col_spec, _col_spec],
    out_specs=(_row_spec, _col_spec),
    out_shape=(
        jax.ShapeDtypeStruct((N_NODES, D), jnp.float32),
        jax.ShapeDtypeStruct((N_NODES, 1), jnp.float32),
    ),
)


def _layer2_body(p0_ref, p1_ref, g1_ref, dinv_ref, w_ref, b_ref, g2_ref):
    pre = dinv_ref[...] * (p0_ref[...] + p1_ref[...] + g1_ref[...]) + b_ref[...]
    x1 = pre * 0.5 * (1.0 + lax.erf(pre * 0.7071067811865476))
    h2 = jnp.dot(x1, w_ref[...], preferred_element_type=jnp.float32)
    g2_ref[...] = h2 * dinv_ref[...]


_layer2 = pl.pallas_call(
    _layer2_body,
    grid=(_NB,),
    in_specs=[_row_spec, _row_spec, _row_spec, _col_spec, _w_spec, _b_spec],
    out_specs=_row_spec,
    out_shape=jax.ShapeDtypeStruct((N_NODES, D), jnp.float32),
)


def _final_body(q0_ref, q1_ref, g2_ref, dinv_ref, b_ref, out_ref):
    out_ref[...] = (
        dinv_ref[...] * (q0_ref[...] + q1_ref[...] + g2_ref[...]) + b_ref[...]
    )


_final = pl.pallas_call(
    _final_body,
    grid=(_NB,),
    in_specs=[_row_spec, _row_spec, _row_spec, _col_spec, _b_spec],
    out_specs=_row_spec,
    out_shape=jax.ShapeDtypeStruct((N_NODES, D), jnp.float32),
)


def kernel(x, edge_index, W1, b1, W2, b2):
    ei = edge_index.astype(jnp.int32)
    npad = EPAD - N_EDGES
    src = jnp.concatenate([ei[0], jnp.zeros((npad,), jnp.int32)])
    dst = jnp.concatenate([ei[1], jnp.full((npad,), NP - 1, jnp.int32)])
    src2 = src.reshape(NCHT, CH)
    dst2 = dst.reshape(NCHT, CH)
    dst4 = dst.reshape(NC, NS, DCH, CH)
    zeros1 = jnp.zeros((RPT,), jnp.float32)
    ones = jnp.ones((CH,), jnp.float32)
    zeros2 = jnp.zeros((RPT, D), jnp.float32)

    deg_p = _deg(dst4, zeros1, ones)
    deg0 = deg_p[0, :N_NODES].reshape(N_NODES, 1)
    deg1 = deg_p[1, :N_NODES].reshape(N_NODES, 1)

    g1, dinv = _mm_scale(x, W1, deg0, deg1)
    acc1 = _gather_scatter(src2, dst2, g1, zeros2)
    g2 = _layer2(acc1[0, :N_NODES], acc1[1, :N_NODES], g1, dinv,
                 W2, b1.reshape(1, D))
    acc2 = _gather_scatter(src2, dst2, g2, zeros2)
    out = _final(acc2[0, :N_NODES], acc2[1, :N_NODES], g2, dinv,
                 b2.reshape(1, D))
    return out


# speedup vs baseline: 2.8340x; 2.8340x over previous
import functools

import jax
import jax.numpy as jnp
from jax import lax
from jax.experimental import pallas as pl
from jax.experimental.pallas import tpu as pltpu
from jax.experimental.pallas import tpu_sc as plsc

N_NODES = 10000
N_EDGES = 320000
D = 128

NC = 2
NS = 16
NW = NC * NS
NP = 10240
RPT = NP // NS
CH = 80
EPW = N_EDGES // NW
NCH = EPW // CH
DCH_W = 128
EPAD = 327680
DCH = EPAD // (NC * NS) // DCH_W

_mesh = plsc.VectorSubcoreMesh(core_axis_name="c", subcore_axis_name="s")


@functools.partial(
    pl.kernel,
    out_type=jax.ShapeDtypeStruct((NC, NP), jnp.float32),
    mesh=_mesh,
    scratch_types=[
        pltpu.VMEM((DCH, DCH_W), jnp.int32),
        pltpu.VMEM((DCH_W,), jnp.float32),
        pltpu.VMEM((RPT,), jnp.float32),
        pltpu.VMEM_SHARED((NP,), jnp.float32),
        pltpu.SemaphoreType.DMA,
    ],
)
def _deg(dst4_hbm, zeros1_hbm, ones_hbm, deg_hbm, dstall, onesv, degv, deg_sh,
         sem):
    c = lax.axis_index("c")
    s = lax.axis_index("s")
    base_r = s * RPT
    pltpu.sync_copy(zeros1_hbm, degv)
    pltpu.sync_copy(degv, deg_sh.at[pl.ds(base_r, RPT)])
    pltpu.sync_copy(ones_hbm, onesv)
    pltpu.sync_copy(dst4_hbm.at[c, s], dstall)
    plsc.subcore_barrier()

    k = 8

    def group(g, carry):
        for b in range(k):
            pltpu.async_copy(onesv, deg_sh.at[dstall.at[g * k + b]], sem,
                             add=True)
        for b in range(k):
            pltpu.make_async_copy(onesv, deg_sh.at[dstall.at[g * k + b]],
                                  sem).wait()
        return carry

    lax.fori_loop(0, DCH // k, group, 0)
    plsc.subcore_barrier()

    pltpu.sync_copy(deg_sh.at[pl.ds(base_r, RPT)], degv)
    pltpu.sync_copy(degv, deg_hbm.at[c, pl.ds(base_r, RPT)])


@functools.partial(
    pl.kernel,
    out_type=jax.ShapeDtypeStruct((NC, NP, D), jnp.float32),
    mesh=_mesh,
    scratch_types=[
        pltpu.VMEM((CH,), jnp.int32),
        pltpu.VMEM((CH,), jnp.int32),
        pltpu.VMEM((CH,), jnp.int32),
        pltpu.VMEM((CH,), jnp.int32),
        pltpu.VMEM((CH, D), jnp.float32),
        pltpu.VMEM((CH, D), jnp.float32),
        pltpu.VMEM_SHARED((NP, D), jnp.float32),
        pltpu.SemaphoreType.DMA,
        pltpu.SemaphoreType.DMA,
    ],
)
def _gather_scatter(src_hbm, dst_hbm, g_hbm, zeros2_hbm, out_hbm,
                    srcv0, srcv1, dstv0, dstv1, rows0, rows1, acc_sh,
                    sg0, sg1):
    c = lax.axis_index("c")
    s = lax.axis_index("s")
    base_r = s * RPT
    base_e = (s * NC + c) * EPW
    srcs = (srcv0, srcv1)
    dsts = (dstv0, dstv1)
    rows = (rows0, rows1)
    sgs = (sg0, sg1)

    def load_idx(i, a):
        off = base_e + i * CH
        pltpu.sync_copy(src_hbm.at[pl.ds(off, CH)], srcs[a])
        pltpu.sync_copy(dst_hbm.at[pl.ds(off, CH)], dsts[a])

    def start_gather(a):
        pltpu.async_copy(g_hbm.at[srcs[a]], rows[a], sgs[a])

    def wait_gather(a):
        pltpu.make_async_copy(g_hbm.at[srcs[a]], rows[a], sgs[a]).wait()

    load_idx(0, 0)
    start_gather(0)
    pltpu.sync_copy(zeros2_hbm, acc_sh.at[pl.ds(base_r, RPT)])
    plsc.subcore_barrier()

    def step(j, a):
        load_idx(j + 1, 1 - a)
        wait_gather(a)
        start_gather(1 - a)
        pltpu.sync_copy(rows[a], acc_sh.at[dsts[a]], add=True)

    def pair(g, carry):
        step(2 * g, 0)
        step(2 * g + 1, 1)
        return carry

    lax.fori_loop(0, (NCH - 1) // 2, pair, 0)
    wait_gather(0)
    pltpu.sync_copy(rows0, acc_sh.at[dstv0], add=True)
    plsc.subcore_barrier()

    pltpu.sync_copy(acc_sh.at[pl.ds(base_r, RPT)], out_hbm.at[c, pl.ds(base_r, RPT)])


_RB = 2000
_NB = N_NODES // _RB

_row_spec = pl.BlockSpec((_RB, D), lambda i: (i, 0))
_col_spec = pl.BlockSpec((_RB, 1), lambda i: (i, 0))
_w_spec = pl.BlockSpec((D, D), lambda i: (0, 0))
_b_spec = pl.BlockSpec((1, D), lambda i: (0, 0))


def _mm_scale_body(x_ref, w_ref, deg0_ref, deg1_ref, g_ref, dinv_ref):
    dinv = lax.rsqrt(deg0_ref[...] + deg1_ref[...] + 1.0)
    dinv_ref[...] = dinv
    h = jnp.dot(x_ref[...], w_ref[...], preferred_element_type=jnp.float32)
    g_ref[...] = h * dinv


_mm_scale = pl.pallas_call(
    _mm_scale_body,
    grid=(_NB,),
    in_specs=[_row_spec, _w_spec, _col_spec, _col_spec],
    out_specs=(_row_spec, _col_spec),
    out_shape=(
        jax.ShapeDtypeStruct((N_NODES, D), jnp.float32),
        jax.ShapeDtypeStruct((N_NODES, 1), jnp.float32),
    ),
)


def _layer2_body(p0_ref, p1_ref, g1_ref, dinv_ref, w_ref, b_ref, g2_ref):
    pre = dinv_ref[...] * (p0_ref[...] + p1_ref[...] + g1_ref[...]) + b_ref[...]
    x1 = pre * 0.5 * (1.0 + lax.erf(pre * 0.7071067811865476))
    h2 = jnp.dot(x1, w_ref[...], preferred_element_type=jnp.float32)
    g2_ref[...] = h2 * dinv_ref[...]


_layer2 = pl.pallas_call(
    _layer2_body,
    grid=(_NB,),
    in_specs=[_row_spec, _row_spec, _row_spec, _col_spec, _w_spec, _b_spec],
    out_specs=_row_spec,
    out_shape=jax.ShapeDtypeStruct((N_NODES, D), jnp.float32),
)


def _final_body(q0_ref, q1_ref, g2_ref, dinv_ref, b_ref, out_ref):
    out_ref[...] = (
        dinv_ref[...] * (q0_ref[...] + q1_ref[...] + g2_ref[...]) + b_ref[...]
    )


_final = pl.pallas_call(
    _final_body,
    grid=(_NB,),
    in_specs=[_row_spec, _row_spec, _row_spec, _col_spec, _b_spec],
    out_specs=_row_spec,
    out_shape=jax.ShapeDtypeStruct((N_NODES, D), jnp.float32),
)


def kernel(x, edge_index, W1, b1, W2, b2):
    ei = edge_index.astype(jnp.int32)
    npad = EPAD - N_EDGES
    src = jnp.concatenate([ei[0], jnp.zeros((npad,), jnp.int32)])
    dst = jnp.concatenate([ei[1], jnp.full((npad,), NP - 1, jnp.int32)])
    dst4 = dst.reshape(NC, NS, DCH, DCH_W)
    zeros1 = jnp.zeros((RPT,), jnp.float32)
    ones = jnp.ones((DCH_W,), jnp.float32)
    zeros2 = jnp.zeros((RPT, D), jnp.float32)

    deg_p = _deg(dst4, zeros1, ones)
    deg0 = deg_p[0, :N_NODES].reshape(N_NODES, 1)
    deg1 = deg_p[1, :N_NODES].reshape(N_NODES, 1)

    g1, dinv = _mm_scale(x, W1, deg0, deg1)
    acc1 = _gather_scatter(src, dst, g1, zeros2)
    g2 = _layer2(acc1[0, :N_NODES], acc1[1, :N_NODES], g1, dinv,
                 W2, b1.reshape(1, D))
    acc2 = _gather_scatter(src, dst, g2, zeros2)
    out = _final(acc2[0, :N_NODES], acc2[1, :N_NODES], g2, dinv,
                 b2.reshape(1, D))
    return out


# speedup vs baseline: 3.2165x; 1.1349x over previous
import functools

import jax
import jax.numpy as jnp
from jax import lax
from jax.experimental import pallas as pl
from jax.experimental.pallas import tpu as pltpu
from jax.experimental.pallas import tpu_sc as plsc

N_NODES = 10000
N_EDGES = 320000
D = 128

NC = 2
NS = 16
NW = NC * NS
NP = 10240
RPT = NP // NS
CH = 80
EPW = N_EDGES // NW
NCH = EPW // CH
DCH_W = 128
EPAD = 327680
DCH = EPAD // (NC * NS) // DCH_W

_mesh = plsc.VectorSubcoreMesh(core_axis_name="c", subcore_axis_name="s")


@functools.partial(
    pl.kernel,
    out_type=jax.ShapeDtypeStruct((NC, NP), jnp.float32),
    mesh=_mesh,
    scratch_types=[
        pltpu.VMEM((DCH, DCH_W), jnp.int32),
        pltpu.VMEM((DCH_W,), jnp.float32),
        pltpu.VMEM((RPT,), jnp.float32),
        pltpu.VMEM_SHARED((NP,), jnp.float32),
        pltpu.SemaphoreType.DMA,
    ],
)
def _deg(dst4_hbm, zeros1_hbm, ones_hbm, deg_hbm, dstall, onesv, degv, deg_sh,
         sem):
    c = lax.axis_index("c")
    s = lax.axis_index("s")
    base_r = s * RPT
    pltpu.sync_copy(zeros1_hbm, degv)
    pltpu.sync_copy(degv, deg_sh.at[pl.ds(base_r, RPT)])
    pltpu.sync_copy(ones_hbm, onesv)
    pltpu.sync_copy(dst4_hbm.at[c, s], dstall)
    plsc.subcore_barrier()

    k = 8

    def group(g, carry):
        for b in range(k):
            pltpu.async_copy(onesv, deg_sh.at[dstall.at[g * k + b]], sem,
                             add=True)
        for b in range(k):
            pltpu.make_async_copy(onesv, deg_sh.at[dstall.at[g * k + b]],
                                  sem).wait()
        return carry

    lax.fori_loop(0, DCH // k, group, 0)
    plsc.subcore_barrier()

    pltpu.sync_copy(deg_sh.at[pl.ds(base_r, RPT)], degv)
    pltpu.sync_copy(degv, deg_hbm.at[c, pl.ds(base_r, RPT)])


@functools.partial(
    pl.kernel,
    out_type=jax.ShapeDtypeStruct((NC, NP, D), jnp.float32),
    mesh=_mesh,
    scratch_types=[
        pltpu.VMEM((CH,), jnp.int32),
        pltpu.VMEM((CH,), jnp.int32),
        pltpu.VMEM((CH,), jnp.int32),
        pltpu.VMEM((CH,), jnp.int32),
        pltpu.VMEM((CH,), jnp.int32),
        pltpu.VMEM((CH,), jnp.int32),
        pltpu.VMEM((CH,), jnp.int32),
        pltpu.VMEM((CH,), jnp.int32),
        pltpu.VMEM((CH, D), jnp.float32),
        pltpu.VMEM((CH, D), jnp.float32),
        pltpu.VMEM_SHARED((NP, D), jnp.float32),
        pltpu.SemaphoreType.DMA,
        pltpu.SemaphoreType.DMA,
        pltpu.SemaphoreType.DMA,
        pltpu.SemaphoreType.DMA,
        pltpu.SemaphoreType.DMA,
        pltpu.SemaphoreType.DMA,
    ],
)
def _gather_scatter(src_hbm, dst_hbm, g_hbm, zeros2_hbm, out_hbm,
                    srcv0, srcv1, srcv2, srcv3, dstv0, dstv1, dstv2, dstv3,
                    rows0, rows1, acc_sh,
                    si0, si1, si2, si3, sg0, sg1):
    c = lax.axis_index("c")
    s = lax.axis_index("s")
    base_r = s * RPT
    base_e = (s * NC + c) * EPW
    srcs = (srcv0, srcv1, srcv2, srcv3)
    dsts = (dstv0, dstv1, dstv2, dstv3)
    rows = (rows0, rows1)
    sis = (si0, si1, si2, si3)
    sgs = (sg0, sg1)

    def start_idx(j, q):
        off = base_e + j * CH
        pltpu.async_copy(src_hbm.at[pl.ds(off, CH)], srcs[q], sis[q])
        pltpu.async_copy(dst_hbm.at[pl.ds(off, CH)], dsts[q], sis[q])

    def wait_idx(j, q):
        off = base_e + j * CH
        pltpu.make_async_copy(src_hbm.at[pl.ds(off, CH)], srcs[q], sis[q]).wait()
        pltpu.make_async_copy(dst_hbm.at[pl.ds(off, CH)], dsts[q], sis[q]).wait()

    def start_gather(q, r):
        pltpu.async_copy(g_hbm.at[srcs[q]], rows[r], sgs[r])

    def wait_gather(q, r):
        pltpu.make_async_copy(g_hbm.at[srcs[q]], rows[r], sgs[r]).wait()

    start_idx(0, 0)
    start_idx(1, 1)
    start_idx(2, 2)
    pltpu.sync_copy(zeros2_hbm, acc_sh.at[pl.ds(base_r, RPT)])
    plsc.subcore_barrier()
    wait_idx(0, 0)
    start_gather(0, 0)

    def quad(gq, carry):
        for b in range(4):
            j = gq * 4 + b
            q, r = b, b % 2
            wait_gather(q, r)
            wait_idx(j + 1, (b + 1) % 4)
            start_gather((b + 1) % 4, 1 - r)

            @pl.when(j + 3 < NCH)
            def _():
                start_idx(j + 3, (b + 3) % 4)

            pltpu.sync_copy(rows[r], acc_sh.at[dsts[q]], add=True)
        return carry

    lax.fori_loop(0, (NCH - 1) // 4, quad, 0)
    wait_gather(0, 0)
    pltpu.sync_copy(rows[0], acc_sh.at[dsts[0]], add=True)
    plsc.subcore_barrier()

    pltpu.sync_copy(acc_sh.at[pl.ds(base_r, RPT)], out_hbm.at[c, pl.ds(base_r, RPT)])


_RB = 2000
_NB = N_NODES // _RB

_row_spec = pl.BlockSpec((_RB, D), lambda i: (i, 0))
_col_spec = pl.BlockSpec((_RB, 1), lambda i: (i, 0))
_w_spec = pl.BlockSpec((D, D), lambda i: (0, 0))
_b_spec = pl.BlockSpec((1, D), lambda i: (0, 0))


def _mm_scale_body(x_ref, w_ref, deg0_ref, deg1_ref, g_ref, dinv_ref):
    dinv = lax.rsqrt(deg0_ref[...] + deg1_ref[...] + 1.0)
    dinv_ref[...] = dinv
    h = jnp.dot(x_ref[...], w_ref[...], preferred_element_type=jnp.float32)
    g_ref[...] = h * dinv


_mm_scale = pl.pallas_call(
    _mm_scale_body,
    grid=(_NB,),
    in_specs=[_row_spec, _w_spec, _col_spec, _col_spec],
    out_specs=(_row_spec, _col_spec),
    out_shape=(
        jax.ShapeDtypeStruct((N_NODES, D), jnp.float32),
        jax.ShapeDtypeStruct((N_NODES, 1), jnp.float32),
    ),
)


def _layer2_body(p0_ref, p1_ref, g1_ref, dinv_ref, w_ref, b_ref, g2_ref):
    pre = dinv_ref[...] * (p0_ref[...] + p1_ref[...] + g1_ref[...]) + b_ref[...]
    x1 = pre * 0.5 * (1.0 + lax.erf(pre * 0.7071067811865476))
    h2 = jnp.dot(x1, w_ref[...], preferred_element_type=jnp.float32)
    g2_ref[...] = h2 * dinv_ref[...]


_layer2 = pl.pallas_call(
    _layer2_body,
    grid=(_NB,),
    in_specs=[_row_spec, _row_spec, _row_spec, _col_spec, _w_spec, _b_spec],
    out_specs=_row_spec,
    out_shape=jax.ShapeDtypeStruct((N_NODES, D), jnp.float32),
)


def _final_body(q0_ref, q1_ref, g2_ref, dinv_ref, b_ref, out_ref):
    out_ref[...] = (
        dinv_ref[...] * (q0_ref[...] + q1_ref[...] + g2_ref[...]) + b_ref[...]
    )


_final = pl.pallas_call(
    _final_body,
    grid=(_NB,),
    in_specs=[_row_spec, _row_spec, _row_spec, _col_spec, _b_spec],
    out_specs=_row_spec,
    out_shape=jax.ShapeDtypeStruct((N_NODES, D), jnp.float32),
)


def kernel(x, edge_index, W1, b1, W2, b2):
    ei = edge_index.astype(jnp.int32)
    npad = EPAD - N_EDGES
    src = jnp.concatenate([ei[0], jnp.zeros((npad,), jnp.int32)])
    dst = jnp.concatenate([ei[1], jnp.full((npad,), NP - 1, jnp.int32)])
    dst4 = dst.reshape(NC, NS, DCH, DCH_W)
    zeros1 = jnp.zeros((RPT,), jnp.float32)
    ones = jnp.ones((DCH_W,), jnp.float32)
    zeros2 = jnp.zeros((RPT, D), jnp.float32)

    deg_p = _deg(dst4, zeros1, ones)
    deg0 = deg_p[0, :N_NODES].reshape(N_NODES, 1)
    deg1 = deg_p[1, :N_NODES].reshape(N_NODES, 1)

    g1, dinv = _mm_scale(x, W1, deg0, deg1)
    acc1 = _gather_scatter(src, dst, g1, zeros2)
    g2 = _layer2(acc1[0, :N_NODES], acc1[1, :N_NODES], g1, dinv,
                 W2, b1.reshape(1, D))
    acc2 = _gather_scatter(src, dst, g2, zeros2)
    out = _final(acc2[0, :N_NODES], acc2[1, :N_NODES], g2, dinv,
                 b2.reshape(1, D))
    return out


# speedup vs baseline: 3.3341x; 1.0366x over previous
import functools

import jax
import jax.numpy as jnp
from jax import lax
from jax.experimental import pallas as pl
from jax.experimental.pallas import tpu as pltpu
from jax.experimental.pallas import tpu_sc as plsc

N_NODES = 10000
N_EDGES = 320000
D = 128

NC = 2
NS = 16
NW = NC * NS
NP = 10240
RPT = NP // NS
CH = 80
EPW = N_EDGES // NW
NCH = EPW // CH
DCH_W = 80
DCH = N_EDGES // (NC * NS) // DCH_W

_mesh = plsc.VectorSubcoreMesh(core_axis_name="c", subcore_axis_name="s")


@functools.partial(
    pl.kernel,
    out_type=jax.ShapeDtypeStruct((NC, NP), jnp.float32),
    mesh=_mesh,
    scratch_types=[
        pltpu.VMEM((DCH, DCH_W), jnp.int32),
        pltpu.VMEM((DCH_W,), jnp.float32),
        pltpu.VMEM((RPT,), jnp.float32),
        pltpu.VMEM_SHARED((NP,), jnp.float32),
        pltpu.SemaphoreType.DMA,
    ],
)
def _deg(dst4_hbm, zeros1_hbm, ones_hbm, deg_hbm, dstall, onesv, degv, deg_sh,
         sem):
    c = lax.axis_index("c")
    s = lax.axis_index("s")
    base_r = s * RPT
    pltpu.sync_copy(zeros1_hbm, degv)
    pltpu.sync_copy(degv, deg_sh.at[pl.ds(base_r, RPT)])
    pltpu.sync_copy(ones_hbm, onesv)
    pltpu.sync_copy(dst4_hbm.at[c, s], dstall)
    plsc.subcore_barrier()

    k = 5

    def group(g, carry):
        for b in range(k):
            pltpu.async_copy(onesv, deg_sh.at[dstall.at[g * k + b]], sem,
                             add=True)
        for b in range(k):
            pltpu.make_async_copy(onesv, deg_sh.at[dstall.at[g * k + b]],
                                  sem).wait()
        return carry

    lax.fori_loop(0, DCH // k, group, 0)
    plsc.subcore_barrier()

    pltpu.sync_copy(deg_sh.at[pl.ds(base_r, RPT)], degv)
    pltpu.sync_copy(degv, deg_hbm.at[c, pl.ds(base_r, RPT)])


@functools.partial(
    pl.kernel,
    out_type=jax.ShapeDtypeStruct((NC, NP, D), jnp.float32),
    mesh=_mesh,
    scratch_types=[
        pltpu.VMEM((CH,), jnp.int32),
        pltpu.VMEM((CH,), jnp.int32),
        pltpu.VMEM((CH,), jnp.int32),
        pltpu.VMEM((CH,), jnp.int32),
        pltpu.VMEM((CH,), jnp.int32),
        pltpu.VMEM((CH,), jnp.int32),
        pltpu.VMEM((CH,), jnp.int32),
        pltpu.VMEM((CH,), jnp.int32),
        pltpu.VMEM((CH, D), jnp.float32),
        pltpu.VMEM((CH, D), jnp.float32),
        pltpu.VMEM_SHARED((NP, D), jnp.float32),
        pltpu.SemaphoreType.DMA,
        pltpu.SemaphoreType.DMA,
        pltpu.SemaphoreType.DMA,
        pltpu.SemaphoreType.DMA,
        pltpu.SemaphoreType.DMA,
        pltpu.SemaphoreType.DMA,
    ],
)
def _gather_scatter(src_hbm, dst_hbm, g_hbm, zeros2_hbm, out_hbm,
                    srcv0, srcv1, srcv2, srcv3, dstv0, dstv1, dstv2, dstv3,
                    rows0, rows1, acc_sh,
                    si0, si1, si2, si3, sg0, sg1):
    c = lax.axis_index("c")
    s = lax.axis_index("s")
    base_r = s * RPT
    base_e = (s * NC + c) * EPW
    srcs = (srcv0, srcv1, srcv2, srcv3)
    dsts = (dstv0, dstv1, dstv2, dstv3)
    rows = (rows0, rows1)
    sis = (si0, si1, si2, si3)
    sgs = (sg0, sg1)

    def start_idx(j, q):
        off = base_e + j * CH
        pltpu.async_copy(src_hbm.at[pl.ds(off, CH)], srcs[q], sis[q])
        pltpu.async_copy(dst_hbm.at[pl.ds(off, CH)], dsts[q], sis[q])

    def wait_idx(j, q):
        off = base_e + j * CH
        pltpu.make_async_copy(src_hbm.at[pl.ds(off, CH)], srcs[q], sis[q]).wait()
        pltpu.make_async_copy(dst_hbm.at[pl.ds(off, CH)], dsts[q], sis[q]).wait()

    def start_gather(q, r):
        pltpu.async_copy(g_hbm.at[srcs[q]], rows[r], sgs[r])

    def wait_gather(q, r):
        pltpu.make_async_copy(g_hbm.at[srcs[q]], rows[r], sgs[r]).wait()

    start_idx(0, 0)
    start_idx(1, 1)
    start_idx(2, 2)
    pltpu.sync_copy(zeros2_hbm, acc_sh.at[pl.ds(base_r, RPT)])
    plsc.subcore_barrier()
    wait_idx(0, 0)
    start_gather(0, 0)

    def quad(gq, carry):
        for b in range(4):
            j = gq * 4 + b
            q, r = b, b % 2
            wait_gather(q, r)
            wait_idx(j + 1, (b + 1) % 4)
            start_gather((b + 1) % 4, 1 - r)

            @pl.when(j + 3 < NCH)
            def _():
                start_idx(j + 3, (b + 3) % 4)

            pltpu.sync_copy(rows[r], acc_sh.at[dsts[q]], add=True)
        return carry

    lax.fori_loop(0, (NCH - 1) // 4, quad, 0)
    wait_gather(0, 0)
    pltpu.sync_copy(rows[0], acc_sh.at[dsts[0]], add=True)
    plsc.subcore_barrier()

    pltpu.sync_copy(acc_sh.at[pl.ds(base_r, RPT)], out_hbm.at[c, pl.ds(base_r, RPT)])


_RB = 2000
_NB = N_NODES // _RB

_row_spec = pl.BlockSpec((_RB, D), lambda i: (i, 0))
_col_spec = pl.BlockSpec((_RB, 1), lambda i: (i, 0))
_w_spec = pl.BlockSpec((D, D), lambda i: (0, 0))
_b_spec = pl.BlockSpec((1, D), lambda i: (0, 0))


def _mm_scale_body(x_ref, w_ref, deg0_ref, deg1_ref, g_ref, dinv_ref):
    dinv = lax.rsqrt(deg0_ref[...] + deg1_ref[...] + 1.0)
    dinv_ref[...] = dinv
    h = jnp.dot(x_ref[...], w_ref[...], preferred_element_type=jnp.float32)
    g_ref[...] = h * dinv


_mm_scale = pl.pallas_call(
    _mm_scale_body,
    grid=(_NB,),
    in_specs=[_row_spec, _w_spec, _col_spec, _col_spec],
    out_specs=(_row_spec, _col_spec),
    out_shape=(
        jax.ShapeDtypeStruct((N_NODES, D), jnp.float32),
        jax.ShapeDtypeStruct((N_NODES, 1), jnp.float32),
    ),
)


_p0_spec = pl.BlockSpec((1, _RB, D), lambda i: (0, i, 0))
_p1_spec = pl.BlockSpec((1, _RB, D), lambda i: (1, i, 0))


def _layer2_body(p0_ref, p1_ref, g1_ref, dinv_ref, w_ref, b_ref, g2_ref):
    pre = (dinv_ref[...] * (p0_ref[0] + p1_ref[0] + g1_ref[...])
           + b_ref[...])
    x1 = pre * 0.5 * (1.0 + lax.erf(pre * 0.7071067811865476))
    h2 = jnp.dot(x1, w_ref[...], preferred_element_type=jnp.float32)
    g2_ref[...] = h2 * dinv_ref[...]


_layer2 = pl.pallas_call(
    _layer2_body,
    grid=(_NB,),
    in_specs=[_p0_spec, _p1_spec, _row_spec, _col_spec, _w_spec, _b_spec],
    out_specs=_row_spec,
    out_shape=jax.ShapeDtypeStruct((N_NODES, D), jnp.float32),
)


def _final_body(q0_ref, q1_ref, g2_ref, dinv_ref, b_ref, out_ref):
    out_ref[...] = (
        dinv_ref[...] * (q0_ref[0] + q1_ref[0] + g2_ref[...]) + b_ref[...]
    )


_final = pl.pallas_call(
    _final_body,
    grid=(_NB,),
    in_specs=[_p0_spec, _p1_spec, _row_spec, _col_spec, _b_spec],
    out_specs=_row_spec,
    out_shape=jax.ShapeDtypeStruct((N_NODES, D), jnp.float32),
)


def kernel(x, edge_index, W1, b1, W2, b2):
    ei = edge_index.astype(jnp.int32)
    src = ei[0]
    dst = ei[1]
    dst4 = dst.reshape(NC, NS, DCH, DCH_W)
    zeros1 = jnp.zeros((RPT,), jnp.float32)
    ones = jnp.ones((DCH_W,), jnp.float32)
    zeros2 = jnp.zeros((RPT, D), jnp.float32)

    deg_p = _deg(dst4, zeros1, ones)
    deg0 = deg_p[0, :N_NODES].reshape(N_NODES, 1)
    deg1 = deg_p[1, :N_NODES].reshape(N_NODES, 1)

    g1, dinv = _mm_scale(x, W1, deg0, deg1)
    acc1 = _gather_scatter(src, dst, g1, zeros2)
    g2 = _layer2(acc1, acc1, g1, dinv, W2, b1.reshape(1, D))
    acc2 = _gather_scatter(src, dst, g2, zeros2)
    out = _final(acc2, acc2, g2, dinv, b2.reshape(1, D))
    return out
